# baseline (speedup 1.0000x reference)
import functools

import jax
import jax.numpy as jnp
from jax import lax
from jax.experimental import pallas as pl
from jax.experimental.pallas import tpu as pltpu
from jax.experimental.pallas import tpu_sc as plsc

NUM_NODES = 1000000
EMBED_DIM = 64
BATCH = 16384


NB = 32768
GRID = -(-NUM_NODES // NB)
PACK_ROWS = GRID * (NB // 2)


def _pack_body(in_ref, out_ref):
    x = in_ref[...]
    y = jnp.swapaxes(x, 0, 1)
    out_ref[...] = jnp.concatenate([y[: NB // 2], y[NB // 2:]], axis=1)


def _pack(emb_t):
    return pl.pallas_call(
        _pack_body,
        grid=(GRID,),
        in_specs=[pl.BlockSpec((EMBED_DIM, NB), lambda g: (0, g))],
        out_specs=pl.BlockSpec((NB // 2, 128), lambda g: (g, 0)),
        out_shape=jax.ShapeDtypeStruct((PACK_ROWS, 128), jnp.float32),
    )(emb_t)


NUM_WORKERS = 32
PAIRS_PER_WORKER = BATCH // NUM_WORKERS
CHUNK = 128
NUM_CHUNKS = PAIRS_PER_WORKER // CHUNK
LANES = 16


def _sc_body(m_hbm, src_hbm, dst_hbm, out_hbm,
             idx_s, idx_d, off_s, off_d, rows_s, rows_d, out_v, sem_s, sem_d):
    wid = lax.axis_index("s") * 2 + lax.axis_index("c")
    base0 = wid * PAIRS_PER_WORKER

    lane = lax.iota(jnp.int32, LANES)
    perms = [lane ^ sh for sh in (8, 4, 2, 1)]

    for k in range(NUM_CHUNKS):
        base = base0 + k * CHUNK
        pltpu.sync_copy(src_hbm.at[pl.ds(base, CHUNK)], idx_s.at[k])
        pltpu.sync_copy(dst_hbm.at[pl.ds(base, CHUNK)], idx_d.at[k])
        for i in range(CHUNK // LANES):
            sl = pl.ds(i * LANES, LANES)
            s_ids = idx_s[k, sl]
            d_ids = idx_d[k, sl]
            idx_s[k, sl] = (
                lax.shift_left(lax.shift_right_logical(s_ids, 15), 14)
                + jnp.bitwise_and(s_ids, 16383))
            idx_d[k, sl] = (
                lax.shift_left(lax.shift_right_logical(d_ids, 15), 14)
                + jnp.bitwise_and(d_ids, 16383))
            off_s[k, sl] = lax.shift_left(
                jnp.bitwise_and(lax.shift_right_logical(s_ids, 14), 1), 6)
            off_d[k, sl] = lax.shift_left(
                jnp.bitwise_and(lax.shift_right_logical(d_ids, 14), 1), 6)

    def issue(k):
        p = k % 2
        return (pltpu.async_copy(m_hbm.at[idx_s.at[k]], rows_s.at[p],
                                 sem_s.at[p]),
                pltpu.async_copy(m_hbm.at[idx_d.at[k]], rows_d.at[p],
                                 sem_d.at[p]))

    cps = {0: issue(0)}
    for k in range(NUM_CHUNKS):
        if k + 1 < NUM_CHUNKS:
            cps[k + 1] = issue(k + 1)
        for cp in cps.pop(k):
            cp.wait()
        p = k % 2

        def block(g, carry, k=k, p=p):
            so = off_s[k, pl.ds(g * LANES, LANES)]
            do = off_d[k, pl.ds(g * LANES, LANES)]
            res = jnp.zeros((LANES,), jnp.float32)
            for w in range(LANES):
                i = g * LANES + w
                acc = jnp.zeros((LANES,), jnp.float32)
                for c in range(EMBED_DIM // LANES):
                    s = rows_s[p, i, pl.ds(so[w] + c * LANES, LANES)]
                    d = rows_d[p, i, pl.ds(do[w] + c * LANES, LANES)]
                    acc = acc + s * d
                for pp in perms:
                    acc = acc + acc[pp]
                res = jnp.where(lane == w, acc, res)
            out_v[pl.ds(k * CHUNK + g * LANES, LANES)] = res
            return carry

        lax.fori_loop(0, CHUNK // LANES, block, 0)

    pltpu.sync_copy(out_v, out_hbm.at[pl.ds(base0, PAIRS_PER_WORKER)])


def _gather_dot(m, src, dst):
    mesh = plsc.VectorSubcoreMesh(core_axis_name="c", subcore_axis_name="s")
    f = pl.kernel(
        _sc_body,
        out_type=jax.ShapeDtypeStruct((BATCH,), jnp.float32),
        mesh=mesh,
        scratch_types=[
            pltpu.VMEM((NUM_CHUNKS, CHUNK), jnp.int32),
            pltpu.VMEM((NUM_CHUNKS, CHUNK), jnp.int32),
            pltpu.VMEM((NUM_CHUNKS, CHUNK), jnp.int32),
            pltpu.VMEM((NUM_CHUNKS, CHUNK), jnp.int32),
            pltpu.VMEM((2, CHUNK, 128), jnp.float32),
            pltpu.VMEM((2, CHUNK, 128), jnp.float32),
            pltpu.VMEM((PAIRS_PER_WORKER,), jnp.float32),
            pltpu.SemaphoreType.DMA((2,)),
            pltpu.SemaphoreType.DMA((2,)),
        ],
    )
    return f(m, src, dst)


@jax.jit
def kernel(node_pairs, embeddings):
    src = node_pairs[:, 0].astype(jnp.int32)
    dst = node_pairs[:, 1].astype(jnp.int32)
    emb_t = embeddings.T
    m = _pack(emb_t)
    return _gather_dot(m, src, dst)
